# Initial kernel scaffold; baseline (speedup 1.0000x reference)
import jax
import jax.numpy as jnp
from jax.experimental import pallas as pl


def kernel(hidden_states, relevance, W_q_rel, W_k_rel, Wq, Wk, Wv, Wo):
    raise NotImplementedError("write your pallas kernel here")


# speedup vs baseline: 9.3565x; 9.3565x over previous
import functools

import jax
import jax.numpy as jnp
from jax.experimental import pallas as pl
from jax.experimental.pallas import tpu as pltpu

L = 2048
D = 1024
H = 16
DH = 64
DREL = 64
KK = 512
BQ = 256
BK = 256
NB = L // BQ

_INT_MIN = -2147483648
_NEG = -1e30

_DN_TT = (((1,), (1,)), ((), ()))
_DN_NN = (((1,), (0,)), ((), ()))


def _proj_body(hid, rel, wq, wk, wv, wqr, wkr, qh, kh, vh, rq, rk):
    h = hid[...]
    r = rel[...]
    qh[...] = jax.lax.dot_general(h, wq[...], _DN_TT,
                                  preferred_element_type=jnp.float32)
    kh[...] = jax.lax.dot_general(h, wk[...], _DN_TT,
                                  preferred_element_type=jnp.float32)
    vh[...] = jax.lax.dot_general(h, wv[...], _DN_TT,
                                  preferred_element_type=jnp.float32)
    rq[...] = jax.lax.dot_general(r, wqr[...], _DN_TT,
                                  preferred_element_type=jnp.float32)
    rk[...] = jax.lax.dot_general(r, wkr[...], _DN_TT,
                                  preferred_element_type=jnp.float32)


def _flash_body(rq, rk, qh, kh, vh, wo, out, mapped, thr, m, l, acc):
    qb = pl.program_id(0)
    kb = pl.program_id(1)

    @pl.when(kb == 0)
    def _scores_and_threshold():
        scores = jax.lax.dot_general(
            rq[...], rk[...], _DN_TT,
            preferred_element_type=jnp.float32) * (DREL ** -0.5)
        rows = qb * BQ + jax.lax.broadcasted_iota(jnp.int32, (BQ, L), 0)
        cols = jax.lax.broadcasted_iota(jnp.int32, (BQ, L), 1)
        bits = jax.lax.bitcast_convert_type(scores, jnp.int32)
        mp = jnp.where(bits >= 0, bits, bits ^ jnp.int32(0x7FFFFFFF))
        mp = jnp.where(cols <= rows, mp, jnp.int32(_INT_MIN))
        mapped[...] = mp
        cnt = jnp.sum((mp >= 0).astype(jnp.int32), axis=1, keepdims=True)
        t = jnp.where(cnt >= KK, jnp.int32(0), jnp.int32(_INT_MIN))
        for b in range(30, -1, -1):
            cand = t | jnp.int32(1 << b)
            cnt = jnp.sum((mp >= cand).astype(jnp.int32), axis=1,
                          keepdims=True)
            t = jnp.where(cnt >= KK, cand, t)
        thr[...] = jnp.broadcast_to(t, (BQ, 128))
        m[...] = jnp.full((BQ, 128), _NEG, dtype=jnp.float32)
        l[...] = jnp.zeros((BQ, 128), jnp.float32)
        acc[...] = jnp.zeros((BQ, D), jnp.float32)

    @pl.when(kb <= qb)
    def _attend():
        t = thr[...][:, 0:1]
        mp_blk = mapped[:, pl.ds(kb * BK, BK)]
        rows = qb * BQ + jax.lax.broadcasted_iota(jnp.int32, (BQ, BK), 0)
        cols = kb * BK + jax.lax.broadcasted_iota(jnp.int32, (BQ, BK), 1)
        allowed = ((mp_blk >= t) | (cols == rows)) & (cols <= rows)
        bias = jnp.where(allowed, jnp.float32(0.0), _NEG)
        for h in range(H):
            sl = slice(h * DH, (h + 1) * DH)
            s = jax.lax.dot_general(
                qh[:, sl], kh[:, sl], _DN_TT,
                preferred_element_type=jnp.float32) * (DH ** -0.5) + bias
            m_old = m[:, h:h + 1]
            m_new = jnp.maximum(jnp.max(s, axis=1, keepdims=True), m_old)
            p = jnp.exp(s - m_new)
            corr = jnp.exp(m_old - m_new)
            l[:, h:h + 1] = l[:, h:h + 1] * corr + jnp.sum(
                p, axis=1, keepdims=True)
            acc[:, sl] = acc[:, sl] * corr + jax.lax.dot_general(
                p, vh[:, sl], _DN_NN, preferred_element_type=jnp.float32)
            m[:, h:h + 1] = m_new

    @pl.when(kb == qb)
    def _finalize():
        for h in range(H):
            sl = slice(h * DH, (h + 1) * DH)
            acc[:, sl] = acc[:, sl] / l[:, h:h + 1]
        out[...] = jax.lax.dot_general(
            acc[...], wo[...], _DN_TT, preferred_element_type=jnp.float32)


@jax.jit
def _run(hs, rel, wqr, wkr, wq, wk, wv, wo):
    qh, kh, vh, rq, rk = pl.pallas_call(
        _proj_body,
        grid=(NB,),
        in_specs=[
            pl.BlockSpec((BQ, D), lambda i: (i, 0)),
            pl.BlockSpec((BQ, D), lambda i: (i, 0)),
            pl.BlockSpec((D, D), lambda i: (0, 0)),
            pl.BlockSpec((D, D), lambda i: (0, 0)),
            pl.BlockSpec((D, D), lambda i: (0, 0)),
            pl.BlockSpec((DREL, D), lambda i: (0, 0)),
            pl.BlockSpec((DREL, D), lambda i: (0, 0)),
        ],
        out_specs=[
            pl.BlockSpec((BQ, D), lambda i: (i, 0)),
            pl.BlockSpec((BQ, D), lambda i: (i, 0)),
            pl.BlockSpec((BQ, D), lambda i: (i, 0)),
            pl.BlockSpec((BQ, DREL), lambda i: (i, 0)),
            pl.BlockSpec((BQ, DREL), lambda i: (i, 0)),
        ],
        out_shape=[
            jax.ShapeDtypeStruct((L, D), jnp.float32),
            jax.ShapeDtypeStruct((L, D), jnp.float32),
            jax.ShapeDtypeStruct((L, D), jnp.float32),
            jax.ShapeDtypeStruct((L, DREL), jnp.float32),
            jax.ShapeDtypeStruct((L, DREL), jnp.float32),
        ],
    )(hs, rel, wq, wk, wv, wqr, wkr)

    out = pl.pallas_call(
        _flash_body,
        grid=(NB, NB),
        in_specs=[
            pl.BlockSpec((BQ, DREL), lambda i, j: (i, 0)),
            pl.BlockSpec((L, DREL), lambda i, j: (0, 0)),
            pl.BlockSpec((BQ, D), lambda i, j: (i, 0)),
            pl.BlockSpec((BK, D), lambda i, j: (jnp.minimum(i, j), 0)),
            pl.BlockSpec((BK, D), lambda i, j: (jnp.minimum(i, j), 0)),
            pl.BlockSpec((D, D), lambda i, j: (0, 0)),
        ],
        out_specs=pl.BlockSpec((BQ, D), lambda i, j: (i, 0)),
        out_shape=jax.ShapeDtypeStruct((L, D), jnp.float32),
        scratch_shapes=[
            pltpu.VMEM((BQ, L), jnp.int32),
            pltpu.VMEM((BQ, 128), jnp.int32),
            pltpu.VMEM((BQ, 128), jnp.float32),
            pltpu.VMEM((BQ, 128), jnp.float32),
            pltpu.VMEM((BQ, D), jnp.float32),
        ],
    )(rq, rk, qh, kh, vh, wo)
    return out


def kernel(hidden_states, relevance, W_q_rel, W_k_rel, Wq, Wk, Wv, Wo):
    hs = hidden_states.reshape(L, D)
    rel = relevance.reshape(L, D)
    out = _run(hs, rel, W_q_rel, W_k_rel, Wq, Wk, Wv, Wo)
    return out.reshape(1, L, D)


# speedup vs baseline: 20.5089x; 2.1919x over previous
import functools

import jax
import jax.numpy as jnp
from jax.experimental import pallas as pl
from jax.experimental.pallas import tpu as pltpu

L = 2048
D = 1024
H = 16
DH = 64
DREL = 64
KK = 512
BQ = 256
NB = L // BQ

_INT_MIN = -2147483648
_NEG = -1e30

_DN_TT = (((1,), (1,)), ((), ()))
_DN_NN = (((1,), (0,)), ((), ()))


def _proj_body(hid, rel, wq, wk, wv, wqr, wkr, qh, kh, vh, rq, rk):
    h = hid[...]
    r = rel[...]
    qh[...] = jax.lax.dot_general(
        h, wq[...], _DN_TT,
        preferred_element_type=jnp.float32).astype(jnp.bfloat16)
    kh[...] = jax.lax.dot_general(
        h, wk[...], _DN_TT,
        preferred_element_type=jnp.float32).astype(jnp.bfloat16)
    vh[...] = jax.lax.dot_general(
        h, wv[...], _DN_TT,
        preferred_element_type=jnp.float32).astype(jnp.bfloat16)
    rq[...] = jax.lax.dot_general(r, wqr[...], _DN_TT,
                                  preferred_element_type=jnp.float32)
    rk[...] = jax.lax.dot_general(r, wkr[...], _DN_TT,
                                  preferred_element_type=jnp.float32)


def _flash_body(rq, rk, qh, kh, vh, wo, out, acc):
    qb = pl.program_id(0)

    scores = jax.lax.dot_general(
        rq[...], rk[...], _DN_TT,
        preferred_element_type=jnp.float32) * (DREL ** -0.5)
    rows = qb * BQ + jax.lax.broadcasted_iota(jnp.int32, (BQ, L), 0)
    cols = jax.lax.broadcasted_iota(jnp.int32, (BQ, L), 1)
    causal = cols <= rows
    bits = jax.lax.bitcast_convert_type(scores, jnp.int32)
    mp = jnp.where(bits >= 0, bits, bits ^ jnp.int32(0x7FFFFFFF))
    mp = jnp.where(causal, mp, jnp.int32(_INT_MIN))
    cnt = jnp.sum((mp >= 0).astype(jnp.int32), axis=1, keepdims=True)
    t = jnp.where(cnt >= KK, jnp.int32(0), jnp.int32(_INT_MIN))
    for b in range(30, -1, -1):
        cand = t | jnp.int32(1 << b)
        cnt = jnp.sum((mp >= cand).astype(jnp.int32), axis=1, keepdims=True)
        t = jnp.where(cnt >= KK, cand, t)
    allowed = ((mp >= t) | (cols == rows)) & causal
    bias = jnp.where(allowed, jnp.float32(0.0), jnp.float32(_NEG))

    for h in range(H):
        sl = slice(h * DH, (h + 1) * DH)
        s = jax.lax.dot_general(
            qh[:, sl], kh[:, sl], _DN_TT,
            preferred_element_type=jnp.float32) * (DH ** -0.5) + bias
        mx = jnp.max(s, axis=1, keepdims=True)
        p = jnp.exp(s - mx)
        sm = jnp.sum(p, axis=1, keepdims=True)
        o_h = jax.lax.dot_general(
            p.astype(jnp.bfloat16), vh[:, sl], _DN_NN,
            preferred_element_type=jnp.float32)
        acc[:, sl] = o_h / sm
    out[...] = jax.lax.dot_general(
        acc[...].astype(jnp.bfloat16), wo[...], _DN_TT,
        preferred_element_type=jnp.float32)


@jax.jit
def _run(hs, rel, wqr, wkr, wq, wk, wv, wo):
    qh, kh, vh, rq, rk = pl.pallas_call(
        _proj_body,
        grid=(NB,),
        in_specs=[
            pl.BlockSpec((BQ, D), lambda i: (i, 0)),
            pl.BlockSpec((BQ, D), lambda i: (i, 0)),
            pl.BlockSpec((D, D), lambda i: (0, 0)),
            pl.BlockSpec((D, D), lambda i: (0, 0)),
            pl.BlockSpec((D, D), lambda i: (0, 0)),
            pl.BlockSpec((DREL, D), lambda i: (0, 0)),
            pl.BlockSpec((DREL, D), lambda i: (0, 0)),
        ],
        out_specs=[
            pl.BlockSpec((BQ, D), lambda i: (i, 0)),
            pl.BlockSpec((BQ, D), lambda i: (i, 0)),
            pl.BlockSpec((BQ, D), lambda i: (i, 0)),
            pl.BlockSpec((BQ, DREL), lambda i: (i, 0)),
            pl.BlockSpec((BQ, DREL), lambda i: (i, 0)),
        ],
        out_shape=[
            jax.ShapeDtypeStruct((L, D), jnp.bfloat16),
            jax.ShapeDtypeStruct((L, D), jnp.bfloat16),
            jax.ShapeDtypeStruct((L, D), jnp.bfloat16),
            jax.ShapeDtypeStruct((L, DREL), jnp.float32),
            jax.ShapeDtypeStruct((L, DREL), jnp.float32),
        ],
    )(hs, rel, wq, wk, wv, wqr, wkr)

    out = pl.pallas_call(
        _flash_body,
        grid=(NB,),
        in_specs=[
            pl.BlockSpec((BQ, DREL), lambda i: (i, 0)),
            pl.BlockSpec((L, DREL), lambda i: (0, 0)),
            pl.BlockSpec((BQ, D), lambda i: (i, 0)),
            pl.BlockSpec((L, D), lambda i: (0, 0)),
            pl.BlockSpec((L, D), lambda i: (0, 0)),
            pl.BlockSpec((D, D), lambda i: (0, 0)),
        ],
        out_specs=pl.BlockSpec((BQ, D), lambda i: (i, 0)),
        out_shape=jax.ShapeDtypeStruct((L, D), jnp.float32),
        scratch_shapes=[
            pltpu.VMEM((BQ, D), jnp.float32),
        ],
    )(rq, rk, qh, kh, vh, wo.astype(jnp.bfloat16))
    return out


def kernel(hidden_states, relevance, W_q_rel, W_k_rel, Wq, Wk, Wv, Wo):
    hs = hidden_states.reshape(L, D)
    rel = relevance.reshape(L, D)
    out = _run(hs, rel, W_q_rel, W_k_rel, Wq, Wk, Wv, Wo)
    return out.reshape(1, L, D)
